# Initial kernel scaffold; baseline (speedup 1.0000x reference)
import jax
import jax.numpy as jnp
from jax.experimental import pallas as pl


def kernel(x, edge_index, batch, W1, b1, W2, b2, W3, b3, linW, linb):
    raise NotImplementedError("write your pallas kernel here")


# speedup vs baseline: 1.2687x; 1.2687x over previous
import jax
import jax.numpy as jnp
from jax.experimental import pallas as pl

_G = 64
_RATIO = 0.5


def _seg_sum(x, batch, g):
    return jax.ops.segment_sum(x, batch, num_segments=g)


def _topk_m(score, batch, nmask, g):
    n = score.shape[0]
    cnt_all = jnp.zeros((g,), jnp.int32).at[batch].add(1)
    cnt_valid = jnp.zeros((g,), jnp.int32).at[batch].add(nmask.astype(jnp.int32))
    k = jnp.ceil(_RATIO * cnt_valid.astype(jnp.float32)).astype(jnp.int32)
    key = jnp.where(nmask, -score, jnp.inf)
    order = jnp.lexsort((key, batch))
    starts = jnp.cumsum(cnt_all) - cnt_all
    bo = batch[order]
    rank = jnp.arange(n, dtype=jnp.int32) - starts[bo]
    sel = (rank < k[bo]) & nmask[order]
    m = jnp.zeros((n,), bool).at[order].set(sel)
    return m


def _final_linear_kernel(xc_ref, w_ref, b_ref, o_ref):
    o_ref[...] = (
        jnp.dot(xc_ref[...], w_ref[...], preferred_element_type=jnp.float32)
        + b_ref[...]
    )


def kernel(x, edge_index, batch, W1, b1, W2, b2, W3, b3, linW, linb):
    n = x.shape[0]
    g = _G
    row = edge_index[1]
    col = edge_index[0]
    f32 = x.dtype

    indeg = jnp.zeros((n,), f32).at[row].add(1.0)
    deg1 = indeg + 1.0
    dis1 = deg1 ** -0.5
    n1 = dis1[row] * dis1[col]
    xw1 = x @ W1
    h1 = jax.nn.relu(
        jnp.zeros((n, xw1.shape[1]), f32).at[row].add(xw1[col] * n1[:, None])
        + xw1 * (dis1 * dis1)[:, None]
        + b1
    )
    disA = jnp.where(indeg > 0, indeg ** -0.5, 0.0)
    nA = disA[row] * disA[col]
    agg1 = jnp.zeros_like(h1).at[row].add(h1[col] * nA[:, None])
    s1 = jnp.sum(jnp.abs(h1 - agg1), axis=1)

    m1 = _topk_m(s1, batch, jnp.ones((n,), bool), g)
    m1f = m1.astype(f32)
    e1 = m1[row] & m1[col]
    e1f = e1.astype(f32)

    hm1 = jnp.where(m1[:, None], h1, 0.0)
    cnt1 = _seg_sum(m1f[:, None], batch, g)
    x1 = jnp.concatenate(
        [jax.ops.segment_max(hm1, batch, num_segments=g),
         _seg_sum(hm1, batch, g) / jnp.maximum(cnt1, 1.0)], axis=1)

    deg2 = jnp.zeros((n,), f32).at[row].add(e1f) + m1f
    dis2 = jnp.where(deg2 > 0, deg2 ** -0.5, 0.0)
    n2 = dis2[row] * dis2[col] * e1f
    xw2 = h1 @ W2
    h2 = jax.nn.relu(
        jnp.zeros((n, xw2.shape[1]), f32).at[row].add(xw2[col] * n2[:, None])
        + xw2 * (dis2 * dis2 * m1f)[:, None]
        + b2
    )
    degI = deg2 - m1f
    disI = jnp.where(degI > 0, degI ** -0.5, 0.0)
    nI = disI[row] * disI[col] * e1f
    agg2 = jnp.zeros_like(h2).at[row].add(h2[col] * nI[:, None])
    s2 = jnp.sum(jnp.abs(h2 - agg2), axis=1)

    m2 = _topk_m(s2, batch, m1, g)
    m2f = m2.astype(f32)
    e2 = e1 & m2[row] & m2[col]
    e2f = e2.astype(f32)

    hm2 = jnp.where(m2[:, None], h2, 0.0)
    cnt2 = _seg_sum(m2f[:, None], batch, g)
    x2 = jnp.concatenate(
        [jax.ops.segment_max(hm2, batch, num_segments=g),
         _seg_sum(hm2, batch, g) / jnp.maximum(cnt2, 1.0)], axis=1)

    deg3 = jnp.zeros((n,), f32).at[row].add(e2f) + m2f
    dis3 = jnp.where(deg3 > 0, deg3 ** -0.5, 0.0)
    n3 = dis3[row] * dis3[col] * e2f
    xw3 = h2 @ W3
    h3 = jax.nn.relu(
        jnp.zeros((n, xw3.shape[1]), f32).at[row].add(xw3[col] * n3[:, None])
        + xw3 * (dis3 * dis3 * m2f)[:, None]
        + b3
    )
    hm3 = jnp.where(m2[:, None], h3, 0.0)
    x3 = jnp.concatenate(
        [jax.ops.segment_max(hm3, batch, num_segments=g),
         _seg_sum(hm3, batch, g) / jnp.maximum(cnt2, 1.0)], axis=1)

    xc = jax.nn.relu(x1) + jax.nn.relu(x2) + jax.nn.relu(x3)
    out = pl.pallas_call(
        _final_linear_kernel,
        out_shape=jax.ShapeDtypeStruct((g, linW.shape[1]), f32),
    )(xc, linW, linb)
    return out


# speedup vs baseline: 12.0071x; 9.4640x over previous
import jax
import jax.numpy as jnp
from jax import lax
from jax.experimental import pallas as pl
from jax.experimental.pallas import tpu as pltpu
from jax.experimental.pallas import tpu_sc as plsc

_G = 64
_RATIO = 0.5
_N = 10000
_E = 320000
_D = 128
_NPAD = 10240
_C = 128
_EPT = _E // 32
_NCHUNK = (_EPT + _C - 1) // _C
_RPT = _NPAD // 16


def _edgepass_body(vals, rowh, colh, out, colv, rowv, rows, acc, sem):
    c = lax.axis_index("c")
    s = lax.axis_index("s")
    lane = lax.iota(jnp.int32, 16)

    def _z(i, _):
        for j in range(_D // 16):
            rows[i, pl.ds(j * 16, 16)] = jnp.zeros((16,), jnp.float32)
        return 0
    lax.fori_loop(0, _C, _z, 0)
    nfull = _RPT // _C
    for r in range(nfull):
        pltpu.sync_copy(rows, acc.at[pl.ds(s * _RPT + r * _C, _C)])
    rem = _RPT - nfull * _C
    if rem:
        pltpu.sync_copy(rows.at[pl.ds(0, rem)],
                        acc.at[pl.ds(s * _RPT + nfull * _C, rem)])
    plsc.subcore_barrier()

    base = (c * 16 + s) * _EPT

    def _chunk(t, _):
        off = base + t * _C
        pltpu.sync_copy(rowh.at[pl.ds(off, _C)], rowv)
        pltpu.sync_copy(colh.at[pl.ds(off, _C)], colv)
        for j in range(_C // 16):
            pos = t * _C + j * 16 + lane
            valid = pos < _EPT
            rv = rowv[pl.ds(j * 16, 16)]
            cv = colv[pl.ds(j * 16, 16)]
            rowv[pl.ds(j * 16, 16)] = jnp.where(valid, rv, _N)
            colv[pl.ds(j * 16, 16)] = jnp.where(valid, cv, 0)
        pltpu.async_copy(vals.at[colv], rows, sem).wait()
        pltpu.sync_copy(rows, acc.at[rowv], add=True)
        return 0

    lax.fori_loop(0, _NCHUNK, _chunk, 0)
    plsc.subcore_barrier()
    pltpu.sync_copy(acc.at[pl.ds(s * _RPT, _RPT)],
                    out.at[c, pl.ds(s * _RPT, _RPT)])


def _edgepass(vals, rowp, colp):
    mesh = plsc.VectorSubcoreMesh(core_axis_name="c", subcore_axis_name="s")
    f = pl.kernel(
        _edgepass_body,
        out_type=jax.ShapeDtypeStruct((2, _NPAD, _D), jnp.float32),
        mesh=mesh,
        scratch_types=[
            pltpu.VMEM((_C,), jnp.int32),
            pltpu.VMEM((_C,), jnp.int32),
            pltpu.VMEM((_C, _D), jnp.float32),
            pltpu.VMEM_SHARED((_NPAD, _D), jnp.float32),
            pltpu.SemaphoreType.DMA,
        ],
    )
    return f(vals, rowp, colp)


def _seg_sum(x, batch, g):
    return jax.ops.segment_sum(x, batch, num_segments=g)


def _topk_m(score, batch, nmask, g):
    n = score.shape[0]
    cnt_all = jnp.zeros((g,), jnp.int32).at[batch].add(1)
    cnt_valid = jnp.zeros((g,), jnp.int32).at[batch].add(nmask.astype(jnp.int32))
    k = jnp.ceil(_RATIO * cnt_valid.astype(jnp.float32)).astype(jnp.int32)
    key = jnp.where(nmask, -score, jnp.inf)
    order = jnp.lexsort((key, batch))
    starts = jnp.cumsum(cnt_all) - cnt_all
    bo = batch[order]
    rank = jnp.arange(n, dtype=jnp.int32) - starts[bo]
    sel = (rank < k[bo]) & nmask[order]
    m = jnp.zeros((n,), bool).at[order].set(sel)
    return m


def _final_linear_kernel(xc_ref, w_ref, b_ref, o_ref):
    o_ref[...] = (
        jnp.dot(xc_ref[...], w_ref[...], preferred_element_type=jnp.float32)
        + b_ref[...]
    )


def kernel(x, edge_index, batch, W1, b1, W2, b2, W3, b3, linW, linb):
    n = x.shape[0]
    g = _G
    row = edge_index[1]
    col = edge_index[0]
    f32 = x.dtype
    epad = _NCHUNK * _C * 32 - _E
    rowp = jnp.concatenate([row, jnp.zeros((epad,), row.dtype)])
    colp = jnp.concatenate([col, jnp.zeros((epad,), col.dtype)])

    def agg_pass(v):
        p = _edgepass(v, rowp, colp)
        return p[0, :n] + p[1, :n]

    indeg = jnp.zeros((n,), f32).at[row].add(1.0)
    deg1 = indeg + 1.0
    dis1 = deg1 ** -0.5
    xw1 = x @ W1
    h1 = jax.nn.relu(
        agg_pass(xw1 * dis1[:, None]) * dis1[:, None]
        + xw1 * (dis1 * dis1)[:, None]
        + b1
    )
    disA = jnp.where(indeg > 0, indeg ** -0.5, 0.0)
    agg1 = agg_pass(h1 * disA[:, None]) * disA[:, None]
    s1 = jnp.sum(jnp.abs(h1 - agg1), axis=1)

    m1 = _topk_m(s1, batch, jnp.ones((n,), bool), g)
    m1f = m1.astype(f32)

    hm1 = jnp.where(m1[:, None], h1, 0.0)
    cnt1 = _seg_sum(m1f[:, None], batch, g)
    x1 = jnp.concatenate(
        [jax.ops.segment_max(hm1, batch, num_segments=g),
         _seg_sum(hm1, batch, g) / jnp.maximum(cnt1, 1.0)], axis=1)

    c2 = agg_pass(jnp.broadcast_to(m1f[:, None], (n, _D)))[:, 0]
    deg2 = m1f * c2 + m1f
    dis2m = m1f * jnp.where(deg2 > 0, deg2 ** -0.5, 0.0)
    xw2 = h1 @ W2
    h2 = jax.nn.relu(
        agg_pass(xw2 * dis2m[:, None]) * dis2m[:, None]
        + xw2 * (dis2m * dis2m)[:, None]
        + b2
    )
    degI = deg2 - m1f
    disIm = m1f * jnp.where(degI > 0, degI ** -0.5, 0.0)
    agg2 = agg_pass(h2 * disIm[:, None]) * disIm[:, None]
    s2 = jnp.sum(jnp.abs(h2 - agg2), axis=1)

    m2 = _topk_m(s2, batch, m1, g)
    m2f = m2.astype(f32)

    hm2 = jnp.where(m2[:, None], h2, 0.0)
    cnt2 = _seg_sum(m2f[:, None], batch, g)
    x2 = jnp.concatenate(
        [jax.ops.segment_max(hm2, batch, num_segments=g),
         _seg_sum(hm2, batch, g) / jnp.maximum(cnt2, 1.0)], axis=1)

    c3 = agg_pass(jnp.broadcast_to(m2f[:, None], (n, _D)))[:, 0]
    deg3 = m2f * c3 + m2f
    dis3m = m2f * jnp.where(deg3 > 0, deg3 ** -0.5, 0.0)
    xw3 = h2 @ W3
    h3 = jax.nn.relu(
        agg_pass(xw3 * dis3m[:, None]) * dis3m[:, None]
        + xw3 * (dis3m * dis3m)[:, None]
        + b3
    )
    hm3 = jnp.where(m2[:, None], h3, 0.0)
    x3 = jnp.concatenate(
        [jax.ops.segment_max(hm3, batch, num_segments=g),
         _seg_sum(hm3, batch, g) / jnp.maximum(cnt2, 1.0)], axis=1)

    xc = jax.nn.relu(x1) + jax.nn.relu(x2) + jax.nn.relu(x3)
    out = pl.pallas_call(
        _final_linear_kernel,
        out_shape=jax.ShapeDtypeStruct((g, linW.shape[1]), f32),
    )(xc, linW, linb)
    return out
